# initial kernel scaffold (unmeasured)
import functools

import jax
import jax.numpy as jnp
from jax import lax
from jax.experimental import pallas as pl
from jax.experimental.pallas import tpu as pltpu

N_DEV = 4
BM = 1024
BN = 512
N_TOTAL = 8192
NBLK = N_TOTAL // BN


def kernel(x, w_mat):
    m_full, k_shard = x.shape
    k_full, n_total = w_mat.shape

    def body(order_ref, x_ref, w_ref, out_ref, recv_ref, acc_ref,
             send_sems, recv_sems):
        j = pl.program_id(0)
        n = pl.program_id(1)
        my = lax.axis_index("i")

        def desc(slot, dev):
            return pltpu.make_async_remote_copy(
                src_ref=x_ref.at[pl.ds(slot * BM, BM), :],
                dst_ref=recv_ref.at[my],
                send_sem=send_sems.at[slot],
                recv_sem=recv_sems.at[slot],
                device_id=(dev,),
                device_id_type=pl.DeviceIdType.MESH,
            )

        @pl.when((j == 0) & (n == 0))
        def _startup():
            barrier = pltpu.get_barrier_semaphore()
            for off in (1, 2, 3):
                pl.semaphore_signal(
                    barrier, inc=1,
                    device_id=((my + off) % N_DEV,),
                    device_id_type=pl.DeviceIdType.MESH,
                )
            pl.semaphore_wait(barrier, 3)
            for off in (1, 3, 2):
                p = (my + off) % N_DEV
                rdma = pltpu.make_async_remote_copy(
                    src_ref=x_ref.at[pl.ds(p * BM, BM), :],
                    dst_ref=recv_ref.at[my],
                    send_sem=send_sems.at[p],
                    recv_sem=recv_sems.at[my],
                    device_id=(p,),
                    device_id_type=pl.DeviceIdType.MESH,
                )
                rdma.start()
            recv_ref[my] = x_ref[pl.ds(my * BM, BM), :]

        kblk = order_ref[j]

        @pl.when((j > 0) & (n == 0))
        def _wait_block():
            desc(kblk, kblk).wait_recv()

        partial = jnp.dot(
            recv_ref[kblk],
            w_ref[:, :],
            preferred_element_type=jnp.float32,
        )
        nsl = pl.ds(n * BN, BN)
        prev = acc_ref[:, nsl]
        total = jnp.where(j == 0, partial, prev + partial)

        @pl.when(j < N_DEV - 1)
        def _acc():
            acc_ref[:, nsl] = total

        @pl.when(j == N_DEV - 1)
        def _emit():
            t = jnp.clip(total, -60.0, 60.0)
            out_ref[:, :] = total / (1.0 + jnp.exp(-t))

        @pl.when((j == N_DEV - 1) & (n == NBLK - 1))
        def _drain_sends():
            for off in (1, 2, 3):
                p = (my + off) % N_DEV
                desc(p, p).wait_send()

    my = lax.axis_index("i")
    order = (my + jnp.array([0, 1, 3, 2], dtype=jnp.int32)) % N_DEV

    grid_spec = pltpu.PrefetchScalarGridSpec(
        num_scalar_prefetch=1,
        grid=(N_DEV, NBLK),
        in_specs=[
            pl.BlockSpec((m_full, k_shard), lambda j, n, ord: (0, 0)),
            pl.BlockSpec((BM, BN), lambda j, n, ord: (ord[j], n)),
        ],
        out_specs=pl.BlockSpec(
            (BM, BN), lambda j, n, ord: (0, jnp.where(j == N_DEV - 1, n, 0))
        ),
        scratch_shapes=[
            pltpu.VMEM((N_DEV, BM, k_shard), jnp.bfloat16),
            pltpu.VMEM((BM, N_TOTAL), jnp.float32),
            pltpu.SemaphoreType.DMA((N_DEV,)),
            pltpu.SemaphoreType.DMA((N_DEV,)),
        ],
    )

    return pl.pallas_call(
        body,
        grid_spec=grid_spec,
        out_shape=jax.ShapeDtypeStruct((BM, n_total), jnp.float32),
        compiler_params=pltpu.CompilerParams(
            collective_id=0,
            dimension_semantics=("arbitrary", "arbitrary"),
        ),
    )(order, x, w_mat)


# baseline (device time: 147827 ns/iter reference)
import jax
import jax.numpy as jnp
from jax import lax
from jax.experimental import pallas as pl
from jax.experimental.pallas import tpu as pltpu

N_DEV = 4
BM = 1024
BN = 512
N_TOTAL = 8192
NBLK = N_TOTAL // BN


def kernel(x, w_mat):
    m_full, k_shard = x.shape
    k_full, n_total = w_mat.shape

    def body(order_ref, x_ref, w_ref, out_ref, xbf_ref, recv_ref, acc_ref,
             send_sems, recv_sems):
        j = pl.program_id(0)
        n = pl.program_id(1)
        my = lax.axis_index("i")

        def send_desc(p):
            return pltpu.make_async_remote_copy(
                src_ref=xbf_ref.at[p],
                dst_ref=recv_ref.at[my],
                send_sem=send_sems.at[p],
                recv_sem=recv_sems.at[my],
                device_id=(p,),
                device_id_type=pl.DeviceIdType.MESH,
            )

        def recv_desc(q):
            return pltpu.make_async_remote_copy(
                src_ref=xbf_ref.at[q],
                dst_ref=recv_ref.at[q],
                send_sem=send_sems.at[q],
                recv_sem=recv_sems.at[q],
                device_id=(q,),
                device_id_type=pl.DeviceIdType.MESH,
            )

        @pl.when((j == 0) & (n == 0))
        def _startup():
            barrier = pltpu.get_barrier_semaphore()
            for off in (1, 2, 3):
                pl.semaphore_signal(
                    barrier, inc=1,
                    device_id=((my + off) % N_DEV,),
                    device_id_type=pl.DeviceIdType.MESH,
                )
            pl.semaphore_wait(barrier, 3)
            for off in (1, 3, 2):
                p = (my + off) % N_DEV
                xbf_ref[p] = x_ref[pl.ds(p * BM, BM), :].astype(jnp.bfloat16)
                send_desc(p).start()
            recv_ref[my] = x_ref[pl.ds(my * BM, BM), :].astype(jnp.bfloat16)

        kblk = order_ref[j]

        @pl.when((j > 0) & (n == 0))
        def _wait_block():
            recv_desc(kblk).wait_recv()

        partial = jnp.dot(
            recv_ref[kblk],
            w_ref[:, :].astype(jnp.bfloat16),
            preferred_element_type=jnp.float32,
        )
        nsl = pl.ds(n * BN, BN)
        prev = acc_ref[:, nsl].astype(jnp.float32)
        total = jnp.where(j == 0, partial, prev + partial)

        @pl.when(j < N_DEV - 1)
        def _acc():
            acc_ref[:, nsl] = total.astype(jnp.bfloat16)

        @pl.when(j == N_DEV - 1)
        def _emit():
            t = jnp.clip(total, -60.0, 60.0)
            out_ref[:, :] = total / (1.0 + jnp.exp(-t))

        @pl.when((j == N_DEV - 1) & (n == NBLK - 1))
        def _drain_sends():
            for off in (1, 2, 3):
                p = (my + off) % N_DEV
                send_desc(p).wait_send()

    my = lax.axis_index("i")
    order = (my + jnp.array([0, 1, 3, 2], dtype=jnp.int32)) % N_DEV

    grid_spec = pltpu.PrefetchScalarGridSpec(
        num_scalar_prefetch=1,
        grid=(N_DEV, NBLK),
        in_specs=[
            pl.BlockSpec((m_full, k_shard), lambda j, n, ord: (0, 0)),
            pl.BlockSpec((BM, BN), lambda j, n, ord: (ord[j], n)),
        ],
        out_specs=pl.BlockSpec(
            (BM, BN), lambda j, n, ord: (0, jnp.where(j == N_DEV - 1, n, 0))
        ),
        scratch_shapes=[
            pltpu.VMEM((N_DEV, BM, k_shard), jnp.bfloat16),
            pltpu.VMEM((N_DEV, BM, k_shard), jnp.bfloat16),
            pltpu.VMEM((BM, N_TOTAL), jnp.bfloat16),
            pltpu.SemaphoreType.DMA((N_DEV,)),
            pltpu.SemaphoreType.DMA((N_DEV,)),
        ],
    )

    return pl.pallas_call(
        body,
        grid_spec=grid_spec,
        out_shape=jax.ShapeDtypeStruct((BM, n_total), jnp.float32),
        compiler_params=pltpu.CompilerParams(
            collective_id=0,
            dimension_semantics=("arbitrary", "arbitrary"),
            vmem_limit_bytes=100 * 1024 * 1024,
        ),
    )(order, x, w_mat)
